# baseline (speedup 1.0000x reference)
import functools

import jax
import jax.numpy as jnp
from jax import lax
from jax.experimental import pallas as pl
from jax.experimental.pallas import tpu as pltpu
from jax.experimental.pallas import tpu_sc as plsc

BATCH = 32
SEQ = 2048
D = 128
NUM_WORKERS = 32
CHUNK = SEQ // NUM_WORKERS
VECS = D // 16
GROUP = 4
ROWS = GROUP * CHUNK
IPG = 128
NGATH = ROWS // IPG
NSTEP = BATCH // GROUP
NBUF = 3


@functools.partial(
    pl.kernel,
    mesh=plsc.VectorSubcoreMesh(core_axis_name="c", subcore_axis_name="s"),
    out_type=jax.ShapeDtypeStruct((BATCH, SEQ, D), jnp.float32),
    scratch_types=(
        [pltpu.VMEM((NGATH, IPG), jnp.int32) for _ in range(NBUF)]
        + [pltpu.VMEM((CHUNK, D), jnp.float32)]
        + [pltpu.VMEM((ROWS, D), jnp.float32) for _ in range(NBUF)]
        + [pltpu.SemaphoreType.DMA for _ in range(3 * NBUF)]
    ),
)
def _emb_kernel(idx_hbm, table_hbm, pos_hbm, out_hbm, *refs):
    idx_v = refs[0:NBUF]
    pos_v = refs[NBUF]
    rows_v = refs[NBUF + 1:2 * NBUF + 1]
    gsem = refs[2 * NBUF + 1:3 * NBUF + 1]
    wsem = refs[3 * NBUF + 1:4 * NBUF + 1]
    isem = refs[4 * NBUF + 1:5 * NBUF + 1]

    c = lax.axis_index("c")
    s = lax.axis_index("s")
    wid = s * 2 + c
    base = wid * CHUNK

    def load_idx(g, p, sync=False):
        for j in range(GROUP):
            off = pl.multiple_of((g * GROUP + j) * SEQ + base, CHUNK)
            dst = idx_v[p].at[j * CHUNK // IPG, pl.ds((j * CHUNK) % IPG, CHUNK)]
            if sync:
                pltpu.sync_copy(idx_hbm.at[pl.ds(off, CHUNK)], dst)
            else:
                pltpu.async_copy(idx_hbm.at[pl.ds(off, CHUNK)], dst, isem[p])

    def wait_idx(p):
        for j in range(GROUP):
            pltpu.make_async_copy(
                idx_hbm.at[pl.ds(0, CHUNK)],
                idx_v[p].at[0, pl.ds(0, CHUNK)], isem[p]).wait()

    def start_gathers(p):
        for k in range(NGATH):
            pltpu.async_copy(table_hbm.at[idx_v[p].at[k]],
                             rows_v[p].at[pl.ds(k * IPG, IPG), :], gsem[p])

    def wait_gathers(p):
        for k in range(NGATH):
            pltpu.make_async_copy(
                table_hbm.at[idx_v[p].at[k]],
                rows_v[p].at[pl.ds(k * IPG, IPG), :], gsem[p]).wait()

    def start_writes(g, p):
        for j in range(GROUP):
            pltpu.async_copy(
                rows_v[p].at[pl.ds(j * CHUNK, CHUNK), :],
                out_hbm.at[g * GROUP + j, pl.ds(base, CHUNK), :], wsem[p])

    def wait_writes(p):
        pltpu.make_async_copy(
            rows_v[p], out_hbm.at[0, pl.ds(0, SEQ), :].at[pl.ds(0, ROWS), :],
            wsem[p]).wait()

    def add_pos(p):
        rows = rows_v[p]

        def add_body(r2, carry):
            for u in range(2):
                r = r2 * 2 + u
                for cc in range(VECS):
                    sl = pl.ds(cc * 16, 16)
                    pv = pos_v[r, sl]
                    for j in range(GROUP):
                        plsc.addupdate(rows.at[j * CHUNK + r, sl], pv)
            return carry

        lax.fori_loop(0, CHUNK // 2, add_body, 0)

    pltpu.sync_copy(pos_hbm.at[pl.ds(base, CHUNK), :], pos_v)

    load_idx(0, 0, sync=True)
    start_gathers(0)
    load_idx(1, 1 % NBUF)

    for g in range(NSTEP):
        p = g % NBUF
        np_ = (g + 1) % NBUF
        if g + 1 < NSTEP:
            wait_idx(np_)
            if g + 1 >= NBUF:
                wait_writes(np_)
            start_gathers(np_)
        wait_gathers(p)
        if g + 2 < NSTEP:
            load_idx(g + 2, (g + 2) % NBUF)
        add_pos(p)
        start_writes(g, p)

    for k in range(max(0, NSTEP - NBUF), NSTEP):
        wait_writes(k % NBUF)


def kernel(input_ids, emb_table, pos_enc):
    out = _emb_kernel(input_ids.astype(jnp.int32).reshape(BATCH * SEQ),
                      emb_table, pos_enc.reshape(SEQ, D))
    return out
